# Initial kernel scaffold; baseline (speedup 1.0000x reference)
import jax
import jax.numpy as jnp
from jax.experimental import pallas as pl


def kernel(item_id, part_id, is_correct, timeliness, elapsed_time_norm, lag_time_norm, shifted_item_id, text_embedding_batch, W_item, W_part, W_correct, W_timeliness, W_elapsed, W_lag, W_shifted_item, pos, W_agg, b_agg):
    raise NotImplementedError("write your pallas kernel here")


# speedup vs baseline: 1.5699x; 1.5699x over previous
import functools

import jax
import jax.numpy as jnp
from jax import lax
from jax.experimental import pallas as pl
from jax.experimental.pallas import tpu as pltpu
from jax.experimental.pallas import tpu_sc as plsc

B, S = 1024, 200
N = B * S
D_ITEM, D_PART, D_SMALL, D_POS, D_MODEL = 64, 16, 16, 32, 256
TOTAL_FEAT = 240

NC, NS = 2, 16
NW = NC * NS
PER_W = N // NW
CHUNK = 640
NCHUNK = PER_W // CHUNK

_sc_mesh = plsc.VectorSubcoreMesh(core_axis_name="c", subcore_axis_name="s")


@functools.partial(
    pl.kernel,
    mesh=_sc_mesh,
    out_type=(
        jax.ShapeDtypeStruct((N, D_ITEM), jnp.float32),
        jax.ShapeDtypeStruct((N, D_ITEM), jnp.float32),
        jax.ShapeDtypeStruct((N, D_PART), jnp.float32),
    ),
    scratch_types=[
        pltpu.VMEM((CHUNK,), jnp.int32),
        pltpu.VMEM((CHUNK,), jnp.int32),
        pltpu.VMEM((CHUNK,), jnp.int32),
        pltpu.VMEM((CHUNK, D_ITEM), jnp.float32),
        pltpu.VMEM((CHUNK, D_ITEM), jnp.float32),
        pltpu.VMEM((CHUNK, D_PART), jnp.float32),
        pltpu.SemaphoreType.DMA,
    ],
    compiler_params=pltpu.CompilerParams(use_tc_tiling_on_sc=False),
)
def _sc_gather(item_idx, shift_idx, part_idx, w_item, w_shift, w_part,
               out_item, out_shift, out_part,
               iidx_v, sidx_v, pidx_v, irows_v, srows_v, prows_v, sem):
    wid = lax.axis_index("s") * NC + lax.axis_index("c")
    base = wid * PER_W

    def body(ci, carry):
        off = base + ci * CHUNK
        pltpu.sync_copy(item_idx.at[pl.ds(off, CHUNK)], iidx_v)
        pltpu.sync_copy(shift_idx.at[pl.ds(off, CHUNK)], sidx_v)
        pltpu.sync_copy(part_idx.at[pl.ds(off, CHUNK)], pidx_v)
        ig = pltpu.async_copy(w_item.at[iidx_v], irows_v, sem)
        sg = pltpu.async_copy(w_shift.at[sidx_v], srows_v, sem)
        pg = pltpu.async_copy(w_part.at[pidx_v], prows_v, sem)
        ig.wait()
        sg.wait()
        pg.wait()
        pltpu.sync_copy(irows_v, out_item.at[pl.ds(off, CHUNK)])
        pltpu.sync_copy(srows_v, out_shift.at[pl.ds(off, CHUNK)])
        pltpu.sync_copy(prows_v, out_part.at[pl.ds(off, CHUNK)])
        return carry

    lax.fori_loop(0, NCHUNK, body, 0)


BT = 8


def _tc_body(gi_ref, gs_ref, gp_ref, ic_ref, it_ref, el_ref, lg_ref,
             pos_ref, small_ref, wagg_ref, bagg_ref, out_ref):
    R = BT * S
    gi = gi_ref[...].reshape(R, D_ITEM)
    gs = gs_ref[...].reshape(R, D_ITEM)
    gp = gp_ref[...].reshape(R, D_PART)
    ic = ic_ref[...].reshape(R, 1)
    it = it_ref[...].reshape(R, 1)
    el = el_ref[...].reshape(R, 1)
    lg = lg_ref[...].reshape(R, 1)
    small = small_ref[...]
    iota3 = lax.broadcasted_iota(jnp.int32, (1, 3), 1)
    sel_c = (ic == iota3).astype(jnp.float32)
    sel_t = (it == iota3).astype(jnp.float32)
    e_corr = jnp.dot(sel_c, small[0:3], preferred_element_type=jnp.float32)
    e_time = jnp.dot(sel_t, small[3:6], preferred_element_type=jnp.float32)
    e_el = el * small[6][None, :]
    e_lg = lg * small[7][None, :]
    posb = jnp.broadcast_to(pos_ref[...][None], (BT, S, D_POS)).reshape(R, D_POS)
    feat = jnp.concatenate([gi, gp, e_corr, e_time, e_el, e_lg, gs, posb], axis=-1)
    acc = lax.dot_general(feat, wagg_ref[...], (((1,), (1,)), ((), ())),
                          preferred_element_type=jnp.float32)
    out_ref[...] = (acc + bagg_ref[...]).reshape(BT, S, D_MODEL)


def _tc_call(g_item, g_shift, g_part, ic3, it3, el3, lg3,
             pos, small, W_agg, b_agg2d):
    blk3 = lambda d: pl.BlockSpec((BT, S, d), lambda i: (i, 0, 0))
    full = lambda shape: pl.BlockSpec(shape, lambda i: (0,) * len(shape))
    return pl.pallas_call(
        _tc_body,
        grid=(B // BT,),
        in_specs=[
            blk3(D_ITEM), blk3(D_ITEM), blk3(D_PART),
            blk3(1), blk3(1), blk3(1), blk3(1),
            full((S, D_POS)), full((8, D_SMALL)),
            full((D_MODEL, TOTAL_FEAT)), full((1, D_MODEL)),
        ],
        out_specs=pl.BlockSpec((BT, S, D_MODEL), lambda i: (i, 0, 0)),
        out_shape=jax.ShapeDtypeStruct((B, S, D_MODEL), jnp.float32),
        compiler_params=pltpu.CompilerParams(
            dimension_semantics=("arbitrary",)),
    )(g_item, g_shift, g_part, ic3, it3, el3, lg3,
      pos, small, W_agg, b_agg2d)


def kernel(item_id, part_id, is_correct, timeliness, elapsed_time_norm,
           lag_time_norm, shifted_item_id, text_embedding_batch,
           W_item, W_part, W_correct, W_timeliness, W_elapsed, W_lag,
           W_shifted_item, pos, W_agg, b_agg):
    item_flat = item_id.reshape(N).astype(jnp.int32)
    shift_flat = shifted_item_id.reshape(N).astype(jnp.int32)
    part_flat = part_id.reshape(N).astype(jnp.int32)

    g_item, g_shift, g_part = _sc_gather(
        item_flat, shift_flat, part_flat, W_item, W_shifted_item, W_part)

    small = jnp.concatenate(
        [W_correct, W_timeliness, W_elapsed.T, W_lag.T], axis=0)

    out = _tc_call(
        g_item.reshape(B, S, D_ITEM),
        g_shift.reshape(B, S, D_ITEM),
        g_part.reshape(B, S, D_PART),
        is_correct.astype(jnp.int32).reshape(B, S, 1),
        timeliness.astype(jnp.int32).reshape(B, S, 1),
        elapsed_time_norm, lag_time_norm,
        pos, small, W_agg, b_agg.reshape(1, D_MODEL))
    return out


# speedup vs baseline: 1.8229x; 1.1612x over previous
import functools

import jax
import jax.numpy as jnp
from jax import lax
from jax.experimental import pallas as pl
from jax.experimental.pallas import tpu as pltpu
from jax.experimental.pallas import tpu_sc as plsc

B, S = 1024, 200
N = B * S
V_ITEM, V_PART = 1000001, 1001
D_ITEM, D_PART, D_SMALL, D_POS, D_MODEL = 64, 16, 16, 32, 256
TOTAL_FEAT = 240

NC, NS = 2, 16
NW = NC * NS
PER_W = N // NW

_sc_mesh = plsc.VectorSubcoreMesh(core_axis_name="c", subcore_axis_name="s")

CHUNK_A = 640
NCHUNK_A = PER_W // CHUNK_A


@functools.partial(
    pl.kernel,
    mesh=_sc_mesh,
    out_type=(
        jax.ShapeDtypeStruct((N, 128), jnp.float32),
        jax.ShapeDtypeStruct((N, 128), jnp.float32),
    ),
    scratch_types=[
        pltpu.VMEM((CHUNK_A,), jnp.int32),
        pltpu.VMEM((CHUNK_A, 128), jnp.float32),
        pltpu.SemaphoreType.DMA,
    ],
)
def _sc_gather_items(item_idx, shift_idx, comb_table, out_item, out_shift,
                     idx_v, rows_v, sem):
    wid = lax.axis_index("s") * NC + lax.axis_index("c")
    base = wid * PER_W

    def pass_over(idx_hbm, out_hbm):
        def body(ci, carry):
            off = base + ci * CHUNK_A
            pltpu.sync_copy(idx_hbm.at[pl.ds(off, CHUNK_A)], idx_v)
            pltpu.async_copy(comb_table.at[idx_v], rows_v, sem).wait()
            pltpu.sync_copy(rows_v, out_hbm.at[pl.ds(off, CHUNK_A)])
            return carry
        lax.fori_loop(0, NCHUNK_A, body, 0)

    pass_over(item_idx, out_item)
    pass_over(shift_idx, out_shift)


CHUNK_B = 1600
NCHUNK_B = PER_W // CHUNK_B


@functools.partial(
    pl.kernel,
    mesh=_sc_mesh,
    out_type=jax.ShapeDtypeStruct((N, D_PART), jnp.float32),
    scratch_types=[
        pltpu.VMEM((CHUNK_B,), jnp.int32),
        pltpu.VMEM((CHUNK_B, D_PART), jnp.float32),
        pltpu.SemaphoreType.DMA,
    ],
    compiler_params=pltpu.CompilerParams(use_tc_tiling_on_sc=False),
)
def _sc_gather_part(part_idx, w_part, out_part, idx_v, rows_v, sem):
    wid = lax.axis_index("s") * NC + lax.axis_index("c")
    base = wid * PER_W

    def body(ci, carry):
        off = base + ci * CHUNK_B
        pltpu.sync_copy(part_idx.at[pl.ds(off, CHUNK_B)], idx_v)
        pltpu.async_copy(w_part.at[idx_v], rows_v, sem).wait()
        pltpu.sync_copy(rows_v, out_part.at[pl.ds(off, CHUNK_B)])
        return carry

    lax.fori_loop(0, NCHUNK_B, body, 0)


BT = 8


def _tc_body(gi_ref, gs_ref, gp_ref, ic_ref, it_ref, el_ref, lg_ref,
             pos_ref, small_ref, wagg_ref, bagg_ref, out_ref):
    R = BT * S
    gi = gi_ref[...][:, :, 0:D_ITEM].reshape(R, D_ITEM)
    gs = gs_ref[...][:, :, D_ITEM:128].reshape(R, D_ITEM)
    gp = gp_ref[...].reshape(R, D_PART)
    ic = ic_ref[...].reshape(R, 1)
    it = it_ref[...].reshape(R, 1)
    el = el_ref[...].reshape(R, 1)
    lg = lg_ref[...].reshape(R, 1)
    small = small_ref[...]
    iota3 = lax.broadcasted_iota(jnp.int32, (1, 3), 1)
    sel_c = (ic == iota3).astype(jnp.float32)
    sel_t = (it == iota3).astype(jnp.float32)
    e_corr = jnp.dot(sel_c, small[0:3], preferred_element_type=jnp.float32)
    e_time = jnp.dot(sel_t, small[3:6], preferred_element_type=jnp.float32)
    e_el = el * small[6][None, :]
    e_lg = lg * small[7][None, :]
    posb = jnp.broadcast_to(pos_ref[...][None], (BT, S, D_POS)).reshape(R, D_POS)
    feat = jnp.concatenate([gi, gp, e_corr, e_time, e_el, e_lg, gs, posb], axis=-1)
    acc = lax.dot_general(feat, wagg_ref[...], (((1,), (1,)), ((), ())),
                          preferred_element_type=jnp.float32)
    out_ref[...] = (acc + bagg_ref[...]).reshape(BT, S, D_MODEL)


def _tc_call(gi128, gs128, g_part, ic3, it3, el3, lg3,
             pos, small, W_agg, b_agg2d):
    blk3 = lambda d: pl.BlockSpec((BT, S, d), lambda i: (i, 0, 0))
    full = lambda shape: pl.BlockSpec(shape, lambda i: (0,) * len(shape))
    return pl.pallas_call(
        _tc_body,
        grid=(B // BT,),
        in_specs=[
            blk3(128), blk3(128), blk3(D_PART),
            blk3(1), blk3(1), blk3(1), blk3(1),
            full((S, D_POS)), full((8, D_SMALL)),
            full((D_MODEL, TOTAL_FEAT)), full((1, D_MODEL)),
        ],
        out_specs=pl.BlockSpec((BT, S, D_MODEL), lambda i: (i, 0, 0)),
        out_shape=jax.ShapeDtypeStruct((B, S, D_MODEL), jnp.float32),
        compiler_params=pltpu.CompilerParams(
            dimension_semantics=("arbitrary",)),
    )(gi128, gs128, g_part, ic3, it3, el3, lg3,
      pos, small, W_agg, b_agg2d)


def kernel(item_id, part_id, is_correct, timeliness, elapsed_time_norm,
           lag_time_norm, shifted_item_id, text_embedding_batch,
           W_item, W_part, W_correct, W_timeliness, W_elapsed, W_lag,
           W_shifted_item, pos, W_agg, b_agg):
    item_flat = item_id.reshape(N).astype(jnp.int32)
    shift_flat = shifted_item_id.reshape(N).astype(jnp.int32)
    part_flat = part_id.reshape(N).astype(jnp.int32)

    comb_table = jnp.concatenate([W_item, W_shifted_item], axis=1)
    gi128, gs128 = _sc_gather_items(item_flat, shift_flat, comb_table)
    g_part = _sc_gather_part(part_flat, W_part)

    small = jnp.concatenate(
        [W_correct, W_timeliness, W_elapsed.T, W_lag.T], axis=0)

    out = _tc_call(
        gi128.reshape(B, S, 128),
        gs128.reshape(B, S, 128),
        g_part.reshape(B, S, D_PART),
        is_correct.astype(jnp.int32).reshape(B, S, 1),
        timeliness.astype(jnp.int32).reshape(B, S, 1),
        elapsed_time_norm, lag_time_norm,
        pos, small, W_agg, b_agg.reshape(1, D_MODEL))
    return out


# speedup vs baseline: 1.8937x; 1.0388x over previous
import functools

import jax
import jax.numpy as jnp
from jax import lax
from jax.experimental import pallas as pl
from jax.experimental.pallas import tpu as pltpu
from jax.experimental.pallas import tpu_sc as plsc

B, S = 1024, 200
N = B * S
V_ITEM, V_PART = 1000001, 1001
D_ITEM, D_PART, D_SMALL, D_POS, D_MODEL = 64, 16, 16, 32, 256
TOTAL_FEAT = 240

NC, NS = 2, 16
NW = NC * NS
PER_W = N // NW
CHUNK = 320
NCHUNK = PER_W // CHUNK
NPAIR = NCHUNK // 2

_sc_mesh = plsc.VectorSubcoreMesh(core_axis_name="c", subcore_axis_name="s")


@functools.partial(
    pl.kernel,
    mesh=_sc_mesh,
    out_type=(
        jax.ShapeDtypeStruct((N, 128), jnp.float32),
        jax.ShapeDtypeStruct((N, 128), jnp.float32),
        jax.ShapeDtypeStruct((N, 128), jnp.float32),
    ),
    scratch_types=[
        pltpu.VMEM((PER_W,), jnp.int32),
        pltpu.VMEM((CHUNK, 128), jnp.float32),
        pltpu.VMEM((CHUNK, 128), jnp.float32),
        pltpu.SemaphoreType.DMA,
        pltpu.SemaphoreType.DMA,
        pltpu.SemaphoreType.DMA,
        pltpu.SemaphoreType.DMA,
    ],
)
def _sc_gather(item_idx, shift_idx, part_idx, comb_table, part_table,
               out_item, out_shift, out_part,
               idx_all, rows0, rows1, g0, g1, w0, w1):
    wid = lax.axis_index("s") * NC + lax.axis_index("c")
    base = wid * PER_W
    rows = (rows0, rows1)
    gsem = (g0, g1)
    wsem = (w0, w1)

    def pass_over(idx_hbm, table, out_hbm):
        pltpu.sync_copy(idx_hbm.at[pl.ds(base, PER_W)], idx_all)

        def start_gather(ci, p):
            idx_sl = idx_all.at[pl.ds(ci * CHUNK, CHUNK)]
            pltpu.async_copy(table.at[idx_sl], rows[p], gsem[p])

        def wait_gather(p):
            pltpu.make_async_copy(
                table.at[pl.ds(0, CHUNK)], rows[p], gsem[p]).wait()

        def start_wb(ci, p):
            pltpu.async_copy(
                rows[p], out_hbm.at[pl.ds(base + ci * CHUNK, CHUNK)], wsem[p])

        def wait_wb(p):
            pltpu.make_async_copy(
                rows[p], out_hbm.at[pl.ds(base, CHUNK)], wsem[p]).wait()

        start_gather(0, 0)

        def body(j, carry):
            wait_gather(0)
            start_gather(2 * j + 1, 1)
            start_wb(2 * j, 0)
            wait_gather(1)
            wait_wb(0)

            @pl.when(j + 1 < NPAIR)
            def _():
                start_gather(2 * j + 2, 0)
            start_wb(2 * j + 1, 1)
            wait_wb(1)
            return carry

        lax.fori_loop(0, NPAIR, body, 0)

    pass_over(item_idx, comb_table, out_item)
    pass_over(shift_idx, comb_table, out_shift)
    pass_over(part_idx, part_table, out_part)


KV = 1024
VPAD = ((V_ITEM + KV - 1) // KV) * KV


def _comb_body(wi_ref, ws_ref, out_ref):
    ii = lax.broadcasted_iota(jnp.int32, (D_ITEM, D_ITEM), 0)
    jj = lax.broadcasted_iota(jnp.int32, (D_ITEM, D_ITEM), 1)
    eye = (ii == jj).astype(jnp.float32)
    cn = (((0,), (0,)), ((), ()))
    ti = lax.dot_general(wi_ref[...], eye, cn,
                         preferred_element_type=jnp.float32)
    ts = lax.dot_general(ws_ref[...], eye, cn,
                         preferred_element_type=jnp.float32)
    out_ref[...] = jnp.concatenate([ti, ts], axis=-1)


def _comb_call(wiT, wsT):
    return pl.pallas_call(
        _comb_body,
        grid=(VPAD // KV,),
        in_specs=[
            pl.BlockSpec((D_ITEM, KV), lambda j: (0, j)),
            pl.BlockSpec((D_ITEM, KV), lambda j: (0, j)),
        ],
        out_specs=pl.BlockSpec((KV, 128), lambda j: (j, 0)),
        out_shape=jax.ShapeDtypeStruct((VPAD, 128), jnp.float32),
        compiler_params=pltpu.CompilerParams(
            dimension_semantics=("arbitrary",)),
    )(wiT, wsT)


BT = 8
RT = BT * S


def _tc_body(gi_ref, gs_ref, gp_ref, ic_ref, it_ref, el_ref, lg_ref,
             pos_ref, small_ref, wagg_ref, bagg_ref, out_ref):
    gi = gi_ref[...][:, 0:D_ITEM]
    gs = gs_ref[...][:, D_ITEM:128]
    gp = gp_ref[...][:, 0:D_PART]
    iota3 = lax.broadcasted_iota(jnp.int32, (1, 1, 3), 2)
    sel_c = (ic_ref[...][:, :, None] == iota3).astype(jnp.float32).reshape(RT, 3)
    sel_t = (it_ref[...][:, :, None] == iota3).astype(jnp.float32).reshape(RT, 3)
    el = el_ref[...].reshape(RT, 1)
    lg = lg_ref[...].reshape(RT, 1)
    small = small_ref[...]
    e_corr = jnp.dot(sel_c, small[0:3], preferred_element_type=jnp.float32)
    e_time = jnp.dot(sel_t, small[3:6], preferred_element_type=jnp.float32)
    e_el = el * small[6][None, :]
    e_lg = lg * small[7][None, :]
    posb = jnp.broadcast_to(pos_ref[...][None], (BT, S, D_POS)).reshape(RT, D_POS)
    feat = jnp.concatenate([gi, gp, e_corr, e_time, e_el, e_lg, gs, posb], axis=-1)
    acc = lax.dot_general(feat, wagg_ref[...], (((1,), (1,)), ((), ())),
                          preferred_element_type=jnp.float32)
    out_ref[...] = (acc + bagg_ref[...]).reshape(BT, S, D_MODEL)


def _tc_call(gi128, gs128, gp128, ic, it, el3, lg3,
             pos, small, W_agg, b_agg2d):
    blkr = pl.BlockSpec((RT, 128), lambda i: (i, 0))
    blk2 = pl.BlockSpec((BT, S), lambda i: (i, 0))
    blk31 = pl.BlockSpec((BT, S, 1), lambda i: (i, 0, 0))
    full = lambda shape: pl.BlockSpec(shape, lambda i: (0,) * len(shape))
    return pl.pallas_call(
        _tc_body,
        grid=(B // BT,),
        in_specs=[
            blkr, blkr, blkr,
            blk2, blk2, blk31, blk31,
            full((S, D_POS)), full((8, D_SMALL)),
            full((D_MODEL, TOTAL_FEAT)), full((1, D_MODEL)),
        ],
        out_specs=pl.BlockSpec((BT, S, D_MODEL), lambda i: (i, 0, 0)),
        out_shape=jax.ShapeDtypeStruct((B, S, D_MODEL), jnp.float32),
        compiler_params=pltpu.CompilerParams(
            dimension_semantics=("arbitrary",)),
    )(gi128, gs128, gp128, ic, it, el3, lg3,
      pos, small, W_agg, b_agg2d)


def kernel(item_id, part_id, is_correct, timeliness, elapsed_time_norm,
           lag_time_norm, shifted_item_id, text_embedding_batch,
           W_item, W_part, W_correct, W_timeliness, W_elapsed, W_lag,
           W_shifted_item, pos, W_agg, b_agg):
    item_flat = item_id.astype(jnp.int32).reshape(N)
    shift_flat = shifted_item_id.astype(jnp.int32).reshape(N)
    part_flat = part_id.astype(jnp.int32).reshape(N)

    comb_table = _comb_call(W_item.T, W_shifted_item.T)
    part_table = jnp.pad(W_part, ((0, 7), (0, 128 - D_PART)))

    gi128, gs128, gp128 = _sc_gather(
        item_flat, shift_flat, part_flat, comb_table, part_table)

    small = jnp.concatenate(
        [W_correct, W_timeliness, W_elapsed.T, W_lag.T], axis=0)

    out = _tc_call(
        gi128, gs128, gp128,
        is_correct.astype(jnp.int32), timeliness.astype(jnp.int32),
        elapsed_time_norm, lag_time_norm,
        pos, small, W_agg, b_agg.reshape(1, D_MODEL))
    return out


# speedup vs baseline: 2.7323x; 1.4429x over previous
import functools

import jax
import jax.numpy as jnp
from jax import lax
from jax.experimental import pallas as pl
from jax.experimental.pallas import tpu as pltpu
from jax.experimental.pallas import tpu_sc as plsc

B, S = 1024, 200
N = B * S
V_ITEM, V_PART = 1000001, 1001
D_ITEM, D_PART, D_SMALL, D_POS, D_MODEL = 64, 16, 16, 32, 256
TOTAL_FEAT = 240

NC, NS = 2, 16
NW = NC * NS
PER_W = N // NW
CHUNK = 400
NCHUNK = PER_W // CHUNK
NPAIR = NCHUNK // 2

_sc_mesh = plsc.VectorSubcoreMesh(core_axis_name="c", subcore_axis_name="s")


@functools.partial(
    pl.kernel,
    mesh=_sc_mesh,
    out_type=(
        jax.ShapeDtypeStruct((N, 128), jnp.float32),
        jax.ShapeDtypeStruct((N, 128), jnp.float32),
    ),
    scratch_types=[
        pltpu.VMEM((PER_W,), jnp.int32),
        pltpu.VMEM((CHUNK, 128), jnp.float32),
        pltpu.VMEM((CHUNK, 128), jnp.float32),
        pltpu.SemaphoreType.DMA,
        pltpu.SemaphoreType.DMA,
        pltpu.SemaphoreType.DMA,
        pltpu.SemaphoreType.DMA,
    ],
)
def _sc_gather(item_idx, shift_idx, comb_table,
               out_item, out_shift,
               idx_all, rows0, rows1, g0, g1, w0, w1):
    wid = lax.axis_index("s") * NC + lax.axis_index("c")
    base = wid * PER_W
    rows = (rows0, rows1)
    gsem = (g0, g1)
    wsem = (w0, w1)

    def pass_over(idx_hbm, table, out_hbm):
        pltpu.sync_copy(idx_hbm.at[pl.ds(base, PER_W)], idx_all)

        def start_gather(ci, p):
            idx_sl = idx_all.at[pl.ds(ci * CHUNK, CHUNK)]
            pltpu.async_copy(table.at[idx_sl], rows[p], gsem[p])

        def wait_gather(p):
            pltpu.make_async_copy(
                table.at[pl.ds(0, CHUNK)], rows[p], gsem[p]).wait()

        def start_wb(ci, p):
            pltpu.async_copy(
                rows[p], out_hbm.at[pl.ds(base + ci * CHUNK, CHUNK)], wsem[p])

        def wait_wb(p):
            pltpu.make_async_copy(
                rows[p], out_hbm.at[pl.ds(base, CHUNK)], wsem[p]).wait()

        start_gather(0, 0)

        def body(j, carry):
            wait_gather(0)
            start_gather(2 * j + 1, 1)
            start_wb(2 * j, 0)
            wait_gather(1)
            wait_wb(0)

            @pl.when(j + 1 < NPAIR)
            def _():
                start_gather(2 * j + 2, 0)
            start_wb(2 * j + 1, 1)
            wait_wb(1)
            return carry

        lax.fori_loop(0, NPAIR, body, 0)

    pass_over(item_idx, comb_table, out_item)
    pass_over(shift_idx, comb_table, out_shift)


CHUNK_P = 1600
NCHUNK_P = PER_W // CHUNK_P
NPAIR_P = NCHUNK_P // 2


@functools.partial(
    pl.kernel,
    mesh=_sc_mesh,
    out_type=jax.ShapeDtypeStruct((N, D_PART), jnp.float32),
    scratch_types=[
        pltpu.VMEM((PER_W,), jnp.int32),
        pltpu.VMEM((CHUNK_P, D_PART), jnp.float32),
        pltpu.VMEM((CHUNK_P, D_PART), jnp.float32),
        pltpu.SemaphoreType.DMA,
        pltpu.SemaphoreType.DMA,
        pltpu.SemaphoreType.DMA,
        pltpu.SemaphoreType.DMA,
    ],
    compiler_params=pltpu.CompilerParams(use_tc_tiling_on_sc=False),
)
def _sc_part(part_idx, part_table, out_part,
             idx_all, rows0, rows1, g0, g1, w0, w1):
    wid = lax.axis_index("s") * NC + lax.axis_index("c")
    base = wid * PER_W
    rows = (rows0, rows1)
    gsem = (g0, g1)
    wsem = (w0, w1)

    pltpu.sync_copy(part_idx.at[pl.ds(base, PER_W)], idx_all)

    def start_gather(ci, p):
        idx_sl = idx_all.at[pl.ds(ci * CHUNK_P, CHUNK_P)]
        pltpu.async_copy(part_table.at[idx_sl], rows[p], gsem[p])

    def wait_gather(p):
        pltpu.make_async_copy(
            part_table.at[pl.ds(0, CHUNK_P)], rows[p], gsem[p]).wait()

    def start_wb(ci, p):
        pltpu.async_copy(
            rows[p], out_part.at[pl.ds(base + ci * CHUNK_P, CHUNK_P)], wsem[p])

    def wait_wb(p):
        pltpu.make_async_copy(
            rows[p], out_part.at[pl.ds(base, CHUNK_P)], wsem[p]).wait()

    start_gather(0, 0)

    def body(j, carry):
        wait_gather(0)
        start_gather(2 * j + 1, 1)
        start_wb(2 * j, 0)
        wait_gather(1)
        wait_wb(0)

        @pl.when(j + 1 < NPAIR_P)
        def _():
            start_gather(2 * j + 2, 0)
        start_wb(2 * j + 1, 1)
        wait_wb(1)
        return carry

    lax.fori_loop(0, NPAIR_P, body, 0)


KV = 4096
VPAD = ((V_ITEM + KV - 1) // KV) * KV


def _comb_body(wi_ref, ws_ref, out_ref):
    ii = lax.broadcasted_iota(jnp.int32, (D_ITEM, D_ITEM), 0)
    jj = lax.broadcasted_iota(jnp.int32, (D_ITEM, D_ITEM), 1)
    eye = (ii == jj).astype(jnp.float32)
    cn = (((0,), (0,)), ((), ()))
    ti = lax.dot_general(wi_ref[...], eye, cn,
                         preferred_element_type=jnp.float32)
    ts = lax.dot_general(ws_ref[...], eye, cn,
                         preferred_element_type=jnp.float32)
    out_ref[...] = jnp.concatenate([ti, ts], axis=-1)


def _comb_call(wiT, wsT):
    return pl.pallas_call(
        _comb_body,
        grid=(VPAD // KV,),
        in_specs=[
            pl.BlockSpec((D_ITEM, KV), lambda j: (0, j)),
            pl.BlockSpec((D_ITEM, KV), lambda j: (0, j)),
        ],
        out_specs=pl.BlockSpec((KV, 128), lambda j: (j, 0)),
        out_shape=jax.ShapeDtypeStruct((VPAD, 128), jnp.float32),
        compiler_params=pltpu.CompilerParams(
            dimension_semantics=("parallel",)),
    )(wiT, wsT)


BT = 16
RT = BT * S


def _tc_body(gi_ref, gs_ref, gp_ref, ic_ref, it_ref, el_ref, lg_ref,
             pos_ref, small_ref, wagg_ref, bagg_ref, out_ref):
    gi = gi_ref[...][:, 0:D_ITEM]
    gs = gs_ref[...][:, D_ITEM:128]
    gp = gp_ref[...]
    iota3 = lax.broadcasted_iota(jnp.int32, (1, 1, 3), 2)
    sel_c = (ic_ref[...][:, :, None] == iota3).astype(jnp.float32).reshape(RT, 3)
    sel_t = (it_ref[...][:, :, None] == iota3).astype(jnp.float32).reshape(RT, 3)
    el = el_ref[...].reshape(RT, 1)
    lg = lg_ref[...].reshape(RT, 1)
    small = small_ref[...]
    e_corr = jnp.dot(sel_c, small[0:3], preferred_element_type=jnp.float32)
    e_time = jnp.dot(sel_t, small[3:6], preferred_element_type=jnp.float32)
    e_el = el * small[6][None, :]
    e_lg = lg * small[7][None, :]
    posb = jnp.broadcast_to(pos_ref[...][None], (BT, S, D_POS)).reshape(RT, D_POS)
    feat = jnp.concatenate([gi, gp, e_corr, e_time, e_el, e_lg, gs, posb], axis=-1)
    acc = lax.dot_general(feat, wagg_ref[...], (((1,), (1,)), ((), ())),
                          preferred_element_type=jnp.float32)
    out_ref[...] = (acc + bagg_ref[...]).reshape(BT, S, D_MODEL)


def _tc_call(gi128, gs128, gp128, ic, it, el3, lg3,
             pos, small, W_agg, b_agg2d):
    blk_i = pl.BlockSpec((RT, 128), lambda i: (i, 0))
    blk_s = pl.BlockSpec((RT, 128), lambda i: (i, 0))
    blk_p = pl.BlockSpec((RT, D_PART), lambda i: (i, 0))
    blk2 = pl.BlockSpec((BT, S), lambda i: (i, 0))
    blk31 = pl.BlockSpec((BT, S, 1), lambda i: (i, 0, 0))
    full = lambda shape: pl.BlockSpec(shape, lambda i: (0,) * len(shape))
    return pl.pallas_call(
        _tc_body,
        grid=(B // BT,),
        in_specs=[
            blk_i, blk_s, blk_p,
            blk2, blk2, blk31, blk31,
            full((S, D_POS)), full((8, D_SMALL)),
            full((D_MODEL, TOTAL_FEAT)), full((1, D_MODEL)),
        ],
        out_specs=pl.BlockSpec((BT, S, D_MODEL), lambda i: (i, 0, 0)),
        out_shape=jax.ShapeDtypeStruct((B, S, D_MODEL), jnp.float32),
        compiler_params=pltpu.CompilerParams(
            dimension_semantics=("arbitrary",)),
    )(gi128, gs128, gp128, ic, it, el3, lg3,
      pos, small, W_agg, b_agg2d)


def kernel(item_id, part_id, is_correct, timeliness, elapsed_time_norm,
           lag_time_norm, shifted_item_id, text_embedding_batch,
           W_item, W_part, W_correct, W_timeliness, W_elapsed, W_lag,
           W_shifted_item, pos, W_agg, b_agg):
    item_flat = item_id.astype(jnp.int32).reshape(N)
    shift_flat = shifted_item_id.astype(jnp.int32).reshape(N)
    part_flat = part_id.astype(jnp.int32).reshape(N)

    gp16 = _sc_part(part_flat, W_part)
    comb_table = _comb_call(W_item.T, W_shifted_item.T)
    gi128, gs128 = _sc_gather(item_flat, shift_flat, comb_table)

    small = jnp.concatenate(
        [W_correct, W_timeliness, W_elapsed.T, W_lag.T], axis=0)

    out = _tc_call(
        gi128, gs128, gp16,
        is_correct.astype(jnp.int32), timeliness.astype(jnp.int32),
        elapsed_time_norm, lag_time_norm,
        pos, small, W_agg, b_agg.reshape(1, D_MODEL))
    return out


# speedup vs baseline: 2.9508x; 1.0800x over previous
import functools

import jax
import jax.numpy as jnp
from jax import lax
from jax.experimental import pallas as pl
from jax.experimental.pallas import tpu as pltpu
from jax.experimental.pallas import tpu_sc as plsc

B, S = 1024, 200
N = B * S
V_ITEM, V_PART = 1000001, 1001
D_ITEM, D_PART, D_SMALL, D_POS, D_MODEL = 64, 16, 16, 32, 256
TOTAL_FEAT = 240

NC, NS = 2, 16
NW = NC * NS
PER_W = N // NW
CHUNK = 400
NCHUNK = PER_W // CHUNK
NPAIR = NCHUNK // 2

_sc_mesh = plsc.VectorSubcoreMesh(core_axis_name="c", subcore_axis_name="s")


@functools.partial(
    pl.kernel,
    mesh=_sc_mesh,
    out_type=(
        jax.ShapeDtypeStruct((N, 128), jnp.float32),
        jax.ShapeDtypeStruct((N, 128), jnp.float32),
    ),
    scratch_types=[
        pltpu.VMEM((PER_W,), jnp.int32),
        pltpu.VMEM((CHUNK, 128), jnp.float32),
        pltpu.VMEM((CHUNK, 128), jnp.float32),
        pltpu.SemaphoreType.DMA,
        pltpu.SemaphoreType.DMA,
        pltpu.SemaphoreType.DMA,
        pltpu.SemaphoreType.DMA,
    ],
)
def _sc_gather(item_idx, shift_idx, comb_table,
               out_item, out_shift,
               idx_all, rows0, rows1, g0, g1, w0, w1):
    wid = lax.axis_index("s") * NC + lax.axis_index("c")
    base = wid * PER_W
    rows = (rows0, rows1)
    gsem = (g0, g1)
    wsem = (w0, w1)

    def pass_over(idx_hbm, table, out_hbm):
        pltpu.sync_copy(idx_hbm.at[pl.ds(base, PER_W)], idx_all)

        def start_gather(ci, p):
            idx_sl = idx_all.at[pl.ds(ci * CHUNK, CHUNK)]
            pltpu.async_copy(table.at[idx_sl], rows[p], gsem[p])

        def wait_gather(p):
            pltpu.make_async_copy(
                table.at[pl.ds(0, CHUNK)], rows[p], gsem[p]).wait()

        def start_wb(ci, p):
            pltpu.async_copy(
                rows[p], out_hbm.at[pl.ds(base + ci * CHUNK, CHUNK)], wsem[p])

        def wait_wb(p):
            pltpu.make_async_copy(
                rows[p], out_hbm.at[pl.ds(base, CHUNK)], wsem[p]).wait()

        start_gather(0, 0)

        def body(j, carry):
            wait_gather(0)
            start_gather(2 * j + 1, 1)
            start_wb(2 * j, 0)
            wait_gather(1)
            wait_wb(0)

            @pl.when(j + 1 < NPAIR)
            def _():
                start_gather(2 * j + 2, 0)
            start_wb(2 * j + 1, 1)
            wait_wb(1)
            return carry

        lax.fori_loop(0, NPAIR, body, 0)

    pass_over(item_idx, comb_table, out_item)
    pass_over(shift_idx, comb_table, out_shift)


CHUNK_P = 1600
NCHUNK_P = PER_W // CHUNK_P
NPAIR_P = NCHUNK_P // 2


@functools.partial(
    pl.kernel,
    mesh=_sc_mesh,
    out_type=jax.ShapeDtypeStruct((N, D_PART), jnp.float32),
    scratch_types=[
        pltpu.VMEM((PER_W,), jnp.int32),
        pltpu.VMEM((CHUNK_P, D_PART), jnp.float32),
        pltpu.VMEM((CHUNK_P, D_PART), jnp.float32),
        pltpu.SemaphoreType.DMA,
        pltpu.SemaphoreType.DMA,
        pltpu.SemaphoreType.DMA,
        pltpu.SemaphoreType.DMA,
    ],
    compiler_params=pltpu.CompilerParams(use_tc_tiling_on_sc=False),
)
def _sc_part(part_idx, part_table, out_part,
             idx_all, rows0, rows1, g0, g1, w0, w1):
    wid = lax.axis_index("s") * NC + lax.axis_index("c")
    base = wid * PER_W
    rows = (rows0, rows1)
    gsem = (g0, g1)
    wsem = (w0, w1)

    pltpu.sync_copy(part_idx.at[pl.ds(base, PER_W)], idx_all)

    def start_gather(ci, p):
        idx_sl = idx_all.at[pl.ds(ci * CHUNK_P, CHUNK_P)]
        pltpu.async_copy(part_table.at[idx_sl], rows[p], gsem[p])

    def wait_gather(p):
        pltpu.make_async_copy(
            part_table.at[pl.ds(0, CHUNK_P)], rows[p], gsem[p]).wait()

    def start_wb(ci, p):
        pltpu.async_copy(
            rows[p], out_part.at[pl.ds(base + ci * CHUNK_P, CHUNK_P)], wsem[p])

    def wait_wb(p):
        pltpu.make_async_copy(
            rows[p], out_part.at[pl.ds(base, CHUNK_P)], wsem[p]).wait()

    start_gather(0, 0)

    def body(j, carry):
        wait_gather(0)
        start_gather(2 * j + 1, 1)
        start_wb(2 * j, 0)
        wait_gather(1)
        wait_wb(0)

        @pl.when(j + 1 < NPAIR_P)
        def _():
            start_gather(2 * j + 2, 0)
        start_wb(2 * j + 1, 1)
        wait_wb(1)
        return carry

    lax.fori_loop(0, NPAIR_P, body, 0)


KV = 8192
VPAD = ((V_ITEM + KV - 1) // KV) * KV


def _comb_body(wi_ref, ws_ref, out_ref):
    ii = lax.broadcasted_iota(jnp.int32, (D_ITEM, D_ITEM), 0)
    jj = lax.broadcasted_iota(jnp.int32, (D_ITEM, D_ITEM), 1)
    eye = (ii == jj).astype(jnp.float32)
    cn = (((0,), (0,)), ((), ()))
    ti = lax.dot_general(wi_ref[...], eye, cn,
                         preferred_element_type=jnp.float32)
    ts = lax.dot_general(ws_ref[...], eye, cn,
                         preferred_element_type=jnp.float32)
    out_ref[...] = jnp.concatenate([ti, ts], axis=-1)


def _comb_call(wiT, wsT):
    return pl.pallas_call(
        _comb_body,
        grid=(VPAD // KV,),
        in_specs=[
            pl.BlockSpec((D_ITEM, KV), lambda j: (0, j)),
            pl.BlockSpec((D_ITEM, KV), lambda j: (0, j)),
        ],
        out_specs=pl.BlockSpec((KV, 128), lambda j: (j, 0)),
        out_shape=jax.ShapeDtypeStruct((VPAD, 128), jnp.float32),
        compiler_params=pltpu.CompilerParams(
            dimension_semantics=("parallel",)),
    )(wiT, wsT)


BT = 32
RT = BT * S


def _tc_body(gi_ref, gs_ref, gp_ref, ic_ref, it_ref, el_ref, lg_ref,
             pos_ref, small_ref, wagg_ref, bagg_ref, out_ref):
    gi = gi_ref[...][:, 0:D_ITEM]
    gs = gs_ref[...][:, D_ITEM:128]
    gp = gp_ref[...]
    iota3 = lax.broadcasted_iota(jnp.int32, (1, 1, 3), 2)
    sel_c = (ic_ref[...][:, :, None] == iota3).astype(jnp.float32).reshape(RT, 3)
    sel_t = (it_ref[...][:, :, None] == iota3).astype(jnp.float32).reshape(RT, 3)
    el = el_ref[...].reshape(RT, 1)
    lg = lg_ref[...].reshape(RT, 1)
    small = small_ref[...]
    e_corr = jnp.dot(sel_c, small[0:3], preferred_element_type=jnp.float32)
    e_time = jnp.dot(sel_t, small[3:6], preferred_element_type=jnp.float32)
    e_el = el * small[6][None, :]
    e_lg = lg * small[7][None, :]
    posb = jnp.broadcast_to(pos_ref[...][None], (BT, S, D_POS)).reshape(RT, D_POS)
    feat = jnp.concatenate([gi, gp, e_corr, e_time, e_el, e_lg, gs, posb], axis=-1)
    acc = lax.dot_general(feat, wagg_ref[...], (((1,), (1,)), ((), ())),
                          preferred_element_type=jnp.float32)
    out_ref[...] = (acc + bagg_ref[...]).reshape(BT, S, D_MODEL)


def _tc_call(gi128, gs128, gp128, ic, it, el3, lg3,
             pos, small, W_agg, b_agg2d):
    blk_i = pl.BlockSpec((RT, 128), lambda i: (i, 0))
    blk_s = pl.BlockSpec((RT, 128), lambda i: (i, 0))
    blk_p = pl.BlockSpec((RT, D_PART), lambda i: (i, 0))
    blk2 = pl.BlockSpec((BT, S), lambda i: (i, 0))
    blk31 = pl.BlockSpec((BT, S, 1), lambda i: (i, 0, 0))
    full = lambda shape: pl.BlockSpec(shape, lambda i: (0,) * len(shape))
    return pl.pallas_call(
        _tc_body,
        grid=(B // BT,),
        in_specs=[
            blk_i, blk_s, blk_p,
            blk2, blk2, blk31, blk31,
            full((S, D_POS)), full((8, D_SMALL)),
            full((D_MODEL, TOTAL_FEAT)), full((1, D_MODEL)),
        ],
        out_specs=pl.BlockSpec((BT, S, D_MODEL), lambda i: (i, 0, 0)),
        out_shape=jax.ShapeDtypeStruct((B, S, D_MODEL), jnp.float32),
        compiler_params=pltpu.CompilerParams(
            dimension_semantics=("arbitrary",)),
    )(gi128, gs128, gp128, ic, it, el3, lg3,
      pos, small, W_agg, b_agg2d)


def kernel(item_id, part_id, is_correct, timeliness, elapsed_time_norm,
           lag_time_norm, shifted_item_id, text_embedding_batch,
           W_item, W_part, W_correct, W_timeliness, W_elapsed, W_lag,
           W_shifted_item, pos, W_agg, b_agg):
    item_flat = item_id.astype(jnp.int32).reshape(N)
    shift_flat = shifted_item_id.astype(jnp.int32).reshape(N)
    part_flat = part_id.astype(jnp.int32).reshape(N)

    gp16 = _sc_part(part_flat, W_part)
    comb_table = _comb_call(W_item.T, W_shifted_item.T)
    gi128, gs128 = _sc_gather(item_flat, shift_flat, comb_table)

    small = jnp.concatenate(
        [W_correct, W_timeliness, W_elapsed.T, W_lag.T], axis=0)

    out = _tc_call(
        gi128, gs128, gp16,
        is_correct.astype(jnp.int32), timeliness.astype(jnp.int32),
        elapsed_time_norm, lag_time_norm,
        pos, small, W_agg, b_agg.reshape(1, D_MODEL))
    return out


# speedup vs baseline: 3.0365x; 1.0290x over previous
import functools

import jax
import jax.numpy as jnp
from jax import lax
from jax.experimental import pallas as pl
from jax.experimental.pallas import tpu as pltpu
from jax.experimental.pallas import tpu_sc as plsc

B, S = 1024, 200
N = B * S
V_ITEM, V_PART = 1000001, 1001
D_ITEM, D_PART, D_SMALL, D_POS, D_MODEL = 64, 16, 16, 32, 256
TOTAL_FEAT = 240

NC, NS = 2, 16
NW = NC * NS
PER_W = N // NW
CHUNK = 400
NCHUNK = PER_W // CHUNK
NPAIR = NCHUNK // 2

_sc_mesh = plsc.VectorSubcoreMesh(core_axis_name="c", subcore_axis_name="s")


@functools.partial(
    pl.kernel,
    mesh=_sc_mesh,
    out_type=(
        jax.ShapeDtypeStruct((N, 128), jnp.float32),
        jax.ShapeDtypeStruct((N, 128), jnp.float32),
    ),
    scratch_types=[
        pltpu.VMEM((PER_W,), jnp.int32),
        pltpu.VMEM((CHUNK, 128), jnp.float32),
        pltpu.VMEM((CHUNK, 128), jnp.float32),
        pltpu.SemaphoreType.DMA,
        pltpu.SemaphoreType.DMA,
        pltpu.SemaphoreType.DMA,
        pltpu.SemaphoreType.DMA,
    ],
)
def _sc_gather(item_idx, shift_idx, comb_table,
               out_item, out_shift,
               idx_all, rows0, rows1, g0, g1, w0, w1):
    wid = lax.axis_index("s") * NC + lax.axis_index("c")
    base = wid * PER_W
    rows = (rows0, rows1)
    gsem = (g0, g1)
    wsem = (w0, w1)

    def pass_over(idx_hbm, table, out_hbm):
        pltpu.sync_copy(idx_hbm.at[pl.ds(base, PER_W)], idx_all)

        def start_gather(ci, p):
            idx_sl = idx_all.at[pl.ds(ci * CHUNK, CHUNK)]
            pltpu.async_copy(table.at[idx_sl], rows[p], gsem[p])

        def wait_gather(p):
            pltpu.make_async_copy(
                table.at[pl.ds(0, CHUNK)], rows[p], gsem[p]).wait()

        def start_wb(ci, p):
            pltpu.async_copy(
                rows[p], out_hbm.at[pl.ds(base + ci * CHUNK, CHUNK)], wsem[p])

        def wait_wb(p):
            pltpu.make_async_copy(
                rows[p], out_hbm.at[pl.ds(base, CHUNK)], wsem[p]).wait()

        start_gather(0, 0)

        def body(j, carry):
            wait_gather(0)
            start_gather(2 * j + 1, 1)
            start_wb(2 * j, 0)
            wait_gather(1)
            wait_wb(0)

            @pl.when(j + 1 < NPAIR)
            def _():
                start_gather(2 * j + 2, 0)
            start_wb(2 * j + 1, 1)
            wait_wb(1)
            return carry

        lax.fori_loop(0, NPAIR, body, 0)

    pass_over(item_idx, comb_table, out_item)
    pass_over(shift_idx, comb_table, out_shift)


CHUNK_P = 1600
NCHUNK_P = PER_W // CHUNK_P
NPAIR_P = NCHUNK_P // 2


@functools.partial(
    pl.kernel,
    mesh=_sc_mesh,
    out_type=jax.ShapeDtypeStruct((N, D_PART), jnp.float32),
    scratch_types=[
        pltpu.VMEM((PER_W,), jnp.int32),
        pltpu.VMEM((CHUNK_P, D_PART), jnp.float32),
        pltpu.VMEM((CHUNK_P, D_PART), jnp.float32),
        pltpu.SemaphoreType.DMA,
        pltpu.SemaphoreType.DMA,
        pltpu.SemaphoreType.DMA,
        pltpu.SemaphoreType.DMA,
    ],
    compiler_params=pltpu.CompilerParams(use_tc_tiling_on_sc=False),
)
def _sc_part(part_idx, part_table, out_part,
             idx_all, rows0, rows1, g0, g1, w0, w1):
    wid = lax.axis_index("s") * NC + lax.axis_index("c")
    base = wid * PER_W
    rows = (rows0, rows1)
    gsem = (g0, g1)
    wsem = (w0, w1)

    pltpu.sync_copy(part_idx.at[pl.ds(base, PER_W)], idx_all)

    def start_gather(ci, p):
        idx_sl = idx_all.at[pl.ds(ci * CHUNK_P, CHUNK_P)]
        pltpu.async_copy(part_table.at[idx_sl], rows[p], gsem[p])

    def wait_gather(p):
        pltpu.make_async_copy(
            part_table.at[pl.ds(0, CHUNK_P)], rows[p], gsem[p]).wait()

    def start_wb(ci, p):
        pltpu.async_copy(
            rows[p], out_part.at[pl.ds(base + ci * CHUNK_P, CHUNK_P)], wsem[p])

    def wait_wb(p):
        pltpu.make_async_copy(
            rows[p], out_part.at[pl.ds(base, CHUNK_P)], wsem[p]).wait()

    start_gather(0, 0)

    def body(j, carry):
        wait_gather(0)
        start_gather(2 * j + 1, 1)
        start_wb(2 * j, 0)
        wait_gather(1)
        wait_wb(0)

        @pl.when(j + 1 < NPAIR_P)
        def _():
            start_gather(2 * j + 2, 0)
        start_wb(2 * j + 1, 1)
        wait_wb(1)
        return carry

    lax.fori_loop(0, NPAIR_P, body, 0)


KV = 16384
VPAD = ((V_ITEM + KV - 1) // KV) * KV


def _comb_body(wi_ref, ws_ref, out_ref):
    ii = lax.broadcasted_iota(jnp.int32, (D_ITEM, D_ITEM), 0)
    jj = lax.broadcasted_iota(jnp.int32, (D_ITEM, D_ITEM), 1)
    eye = (ii == jj).astype(jnp.float32)
    cn = (((0,), (0,)), ((), ()))
    ti = lax.dot_general(wi_ref[...], eye, cn,
                         preferred_element_type=jnp.float32)
    ts = lax.dot_general(ws_ref[...], eye, cn,
                         preferred_element_type=jnp.float32)
    out_ref[...] = jnp.concatenate([ti, ts], axis=-1)


def _comb_call(wiT, wsT):
    return pl.pallas_call(
        _comb_body,
        grid=(VPAD // KV,),
        in_specs=[
            pl.BlockSpec((D_ITEM, KV), lambda j: (0, j)),
            pl.BlockSpec((D_ITEM, KV), lambda j: (0, j)),
        ],
        out_specs=pl.BlockSpec((KV, 128), lambda j: (j, 0)),
        out_shape=jax.ShapeDtypeStruct((VPAD, 128), jnp.float32),
        compiler_params=pltpu.CompilerParams(
            dimension_semantics=("parallel",)),
    )(wiT, wsT)


BT = 32
RT = BT * S


def _tc_body(gi_ref, gs_ref, gp_ref, ic_ref, it_ref, el_ref, lg_ref,
             pos_ref, small_ref, wagg_ref, bagg_ref, out_ref):
    gi = gi_ref[...][:, 0:D_ITEM]
    gs = gs_ref[...][:, D_ITEM:128]
    gp = gp_ref[...]
    iota3 = lax.broadcasted_iota(jnp.int32, (1, 1, 3), 2)
    sel_c = (ic_ref[...][:, :, None] == iota3).astype(jnp.float32).reshape(RT, 3)
    sel_t = (it_ref[...][:, :, None] == iota3).astype(jnp.float32).reshape(RT, 3)
    el = el_ref[...].reshape(RT, 1)
    lg = lg_ref[...].reshape(RT, 1)
    small = small_ref[...]
    e_corr = jnp.dot(sel_c, small[0:3], preferred_element_type=jnp.float32)
    e_time = jnp.dot(sel_t, small[3:6], preferred_element_type=jnp.float32)
    e_el = el * small[6][None, :]
    e_lg = lg * small[7][None, :]
    posb = jnp.broadcast_to(pos_ref[...][None], (BT, S, D_POS)).reshape(RT, D_POS)
    feat = jnp.concatenate([gi, gp, e_corr, e_time, e_el, e_lg, gs, posb], axis=-1)
    acc = lax.dot_general(feat, wagg_ref[...], (((1,), (1,)), ((), ())),
                          preferred_element_type=jnp.float32)
    out_ref[...] = (acc + bagg_ref[...]).reshape(BT, S, D_MODEL)


def _tc_call(gi128, gs128, gp128, ic, it, el3, lg3,
             pos, small, W_agg, b_agg2d):
    blk_i = pl.BlockSpec((RT, 128), lambda i: (i, 0))
    blk_s = pl.BlockSpec((RT, 128), lambda i: (i, 0))
    blk_p = pl.BlockSpec((RT, D_PART), lambda i: (i, 0))
    blk2 = pl.BlockSpec((BT, S), lambda i: (i, 0))
    blk31 = pl.BlockSpec((BT, S, 1), lambda i: (i, 0, 0))
    full = lambda shape: pl.BlockSpec(shape, lambda i: (0,) * len(shape))
    return pl.pallas_call(
        _tc_body,
        grid=(B // BT,),
        in_specs=[
            blk_i, blk_s, blk_p,
            blk2, blk2, blk31, blk31,
            full((S, D_POS)), full((8, D_SMALL)),
            full((D_MODEL, TOTAL_FEAT)), full((1, D_MODEL)),
        ],
        out_specs=pl.BlockSpec((BT, S, D_MODEL), lambda i: (i, 0, 0)),
        out_shape=jax.ShapeDtypeStruct((B, S, D_MODEL), jnp.float32),
        compiler_params=pltpu.CompilerParams(
            dimension_semantics=("arbitrary",)),
    )(gi128, gs128, gp128, ic, it, el3, lg3,
      pos, small, W_agg, b_agg2d)


def kernel(item_id, part_id, is_correct, timeliness, elapsed_time_norm,
           lag_time_norm, shifted_item_id, text_embedding_batch,
           W_item, W_part, W_correct, W_timeliness, W_elapsed, W_lag,
           W_shifted_item, pos, W_agg, b_agg):
    item_flat = item_id.astype(jnp.int32).reshape(N)
    shift_flat = shifted_item_id.astype(jnp.int32).reshape(N)
    part_flat = part_id.astype(jnp.int32).reshape(N)

    gp16 = _sc_part(part_flat, W_part)
    comb_table = _comb_call(W_item.T, W_shifted_item.T)
    gi128, gs128 = _sc_gather(item_flat, shift_flat, comb_table)

    small = jnp.concatenate(
        [W_correct, W_timeliness, W_elapsed.T, W_lag.T], axis=0)

    out = _tc_call(
        gi128, gs128, gp16,
        is_correct.astype(jnp.int32), timeliness.astype(jnp.int32),
        elapsed_time_norm, lag_time_norm,
        pos, small, W_agg, b_agg.reshape(1, D_MODEL))
    return out
